# baseline (speedup 1.0000x reference)
import functools

import jax
import jax.numpy as jnp
from jax import lax
from jax.experimental import pallas as pl
from jax.experimental.pallas import tpu as pltpu
from jax.experimental.pallas import tpu_sc as plsc

N = 10000
D = 128
CHUNK = 128
N_SUBCORES = 16
PHASES = 4
SLAB = 632
N_PAD = SLAB * N_SUBCORES
SLAB_CHUNKS = (128, 128, 128, 128, 120)


def _make_sc_agg(ch_per_tile):
    cpp = ch_per_tile // PHASES
    pairs = cpp // 2
    mesh = plsc.VectorSubcoreMesh(core_axis_name="c", subcore_axis_name="s")

    def body(a_hbm, b_hbm, gr_hbm, sr_hbm, gl_hbm, sl_hbm, zeros_hbm,
             outr_hbm, outl_hbm,
             gidx, sidx, rows, agg, sem0, sem1):
        c = lax.axis_index("c")
        s = lax.axis_index("s")
        r0 = pl.multiple_of(s * SLAB, 8)

        def run_dir(h_hbm, g_hbm, s_hbm, out_hbm):
            pltpu.sync_copy(zeros_hbm, rows.at[0])
            off = 0
            for ck in SLAB_CHUNKS:
                pltpu.sync_copy(rows.at[0, pl.ds(0, ck)],
                                agg.at[pl.ds(r0 + off, ck)])
                off += ck
            plsc.subcore_barrier()

            sems = (sem0, sem1)

            def fire(j, b):
                return pltpu.async_copy(h_hbm.at[gidx.at[j]], rows.at[b],
                                        sems[b])

            def drain_and_scatter(j, b):
                pltpu.make_async_copy(h_hbm.at[gidx.at[j]], rows.at[b],
                                      sems[b]).wait()
                pltpu.sync_copy(rows.at[b], agg.at[sidx.at[j]], add=True)

            def loop_body(jj, carry):
                j = 2 * jj
                fire(j + 1, 1)
                drain_and_scatter(j, 0)

                @pl.when(jj < pairs - 1)
                def _():
                    fire(j + 2, 0)

                drain_and_scatter(j + 1, 1)
                return carry

            for p in range(PHASES):
                ch0 = pl.multiple_of(s * ch_per_tile + p * cpp, 8)
                pltpu.sync_copy(g_hbm.at[pl.ds(ch0, cpp)], gidx)
                pltpu.sync_copy(s_hbm.at[pl.ds(ch0, cpp)], sidx)
                fire(0, 0)
                lax.fori_loop(0, pairs, loop_body, 0, unroll=False)
            plsc.subcore_barrier()

            off = 0
            for ck in SLAB_CHUNKS:
                pltpu.sync_copy(agg.at[pl.ds(r0 + off, ck)],
                                rows.at[0, pl.ds(0, ck)])
                pltpu.sync_copy(rows.at[0, pl.ds(0, ck)],
                                out_hbm.at[pl.ds(r0 + off, ck)])
                off += ck

        @pl.when(c == 0)
        def _():
            run_dir(a_hbm, gr_hbm, sr_hbm, outr_hbm)

        @pl.when(c == 1)
        def _():
            run_dir(b_hbm, gl_hbm, sl_hbm, outl_hbm)

    return pl.kernel(
        body,
        out_type=(jax.ShapeDtypeStruct((N_PAD, D), jnp.float32),
                  jax.ShapeDtypeStruct((N_PAD, D), jnp.float32)),
        mesh=mesh,
        scratch_types=[
            pltpu.VMEM((cpp, CHUNK), jnp.int32),
            pltpu.VMEM((cpp, CHUNK), jnp.int32),
            pltpu.VMEM((2, CHUNK, D), jnp.float32),
            pltpu.VMEM_SHARED((N_PAD, D), jnp.float32),
            pltpu.SemaphoreType.DMA,
            pltpu.SemaphoreType.DMA,
        ],
    )


BLK = 2000
GRID = N // BLK


def _ln_relu(x, g, b):
    m = jnp.mean(x, axis=-1, keepdims=True)
    v = jnp.mean((x - m) ** 2, axis=-1, keepdims=True)
    return jax.nn.relu((x - m) / jnp.sqrt(v + 1e-5) * g + b)


def _dense_body(residual, aggr, aggl, rootr, rootl, left, right,
                wrel_lr, wroot_lr, wrel_rl, wroot_rl, brel_lr, brel_rl,
                gl, bl, gr, br,
                left_o, right_o, hl_o, hr_o):
    r_new = (jnp.dot(aggr[...], wrel_lr[...], preferred_element_type=jnp.float32)
             + brel_lr[...]
             + jnp.dot(rootr[...], wroot_lr[...], preferred_element_type=jnp.float32))
    l_new = (jnp.dot(aggl[...], wrel_rl[...], preferred_element_type=jnp.float32)
             + brel_rl[...]
             + jnp.dot(rootl[...], wroot_rl[...], preferred_element_type=jnp.float32))
    if residual:
        l_new = left[...] + l_new
        r_new = right[...] + r_new
    left_o[...] = l_new
    right_o[...] = r_new
    hl_o[...] = _ln_relu(l_new, gl[...], bl[...])
    hr_o[...] = _ln_relu(r_new, gr[...], br[...])


def _row_spec():
    return pl.BlockSpec((BLK, D), lambda i: (i, 0))


def _w_spec():
    return pl.BlockSpec((D, D), lambda i: (0, 0))


def _v_spec():
    return pl.BlockSpec((1, D), lambda i: (0, 0))


def _make_tc_layer(residual):
    n_row_in = 6 if residual else 4
    in_specs = ([_row_spec()] * n_row_in
                + [_w_spec()] * 4
                + [_v_spec()] * 6)
    body = functools.partial(_dense_body, residual)
    if not residual:
        def body(aggr, aggl, rootr, rootl, *rest):
            return _dense_body(False, aggr, aggl, rootr, rootl, None, None,
                               *rest)
    return pl.pallas_call(
        body,
        grid=(GRID,),
        in_specs=in_specs,
        out_specs=[_row_spec()] * 4,
        out_shape=[jax.ShapeDtypeStruct((N, D), jnp.float32)] * 4,
    )


def _final_body(aggr, aggl, left, right, xs, xt,
                wrel_lr, wroot_lr, wrel_rl, wroot_rl, brel_lr, brel_rl,
                l_o, r_o):
    r_new = (jnp.dot(aggr[...], wrel_lr[...], preferred_element_type=jnp.float32)
             + brel_lr[...]
             + jnp.dot(right[...], wroot_lr[...], preferred_element_type=jnp.float32))
    l_new = (jnp.dot(aggl[...], wrel_rl[...], preferred_element_type=jnp.float32)
             + brel_rl[...]
             + jnp.dot(left[...], wroot_rl[...], preferred_element_type=jnp.float32))

    def knowledge(v, flags):
        nrm = jnp.sqrt(jnp.sum(v * v, axis=-1, keepdims=True))
        v = v / jnp.maximum(nrm, 1e-12) * 10.0
        col = lax.broadcasted_iota(jnp.int32, v.shape, 1)
        lo = (jnp.abs(flags[:, 125:126]) > 0) & (col == 0)
        hi = (jnp.abs(flags[:, 127:128]) > 0) & (col == 2)
        return v - jnp.where(lo | hi, 10.0, 0.0)

    l_o[...] = knowledge(l_new, xs[...])
    r_o[...] = knowledge(r_new, xt[...])


_tc_final = pl.pallas_call(
    _final_body,
    grid=(GRID,),
    in_specs=[_row_spec()] * 6 + [_w_spec()] * 4 + [_v_spec()] * 2,
    out_specs=[_row_spec()] * 2,
    out_shape=[jax.ShapeDtypeStruct((N, D), jnp.float32)] * 2,
)


def kernel(x_s, x_t, edge_index,
           W_rel0_lr, b_rel0_lr, W_root0_lr, W_rel0_rl, b_rel0_rl, W_root0_rl,
           Wrel_lr_mid, brel_lr_mid, Wroot_lr_mid, Wrel_rl_mid, brel_rl_mid,
           Wroot_rl_mid, ln_g_l, ln_b_l, ln_g_r, ln_b_r,
           Wrel_last_lr, brel_last_lr, Wroot_last_lr, Wrel_last_rl,
           brel_last_rl, Wroot_last_rl):
    E = edge_index.shape[1]
    mid = Wrel_lr_mid.shape[0]

    n_chunks = -(-E // CHUNK)
    ch_per_tile = -(-n_chunks // N_SUBCORES)
    align = PHASES * 8
    ch_per_tile = -(-ch_per_tile // align) * align
    pad_e = ch_per_tile * N_SUBCORES * CHUNK

    src = edge_index[0]
    dst = edge_index[1]
    zpad_g = jnp.zeros((pad_e - E,), jnp.int32)
    zpad_s = jnp.full((pad_e - E,), N, jnp.int32)
    shape2 = (pad_e // CHUNK, CHUNK)
    g_r = jnp.concatenate([src, zpad_g]).reshape(shape2)
    s_r = jnp.concatenate([dst, zpad_s]).reshape(shape2)
    g_l = jnp.concatenate([dst, zpad_g]).reshape(shape2)
    s_l = jnp.concatenate([src, zpad_s]).reshape(shape2)
    zeros = jnp.zeros((CHUNK, D), jnp.float32)

    sc_agg = _make_sc_agg(ch_per_tile)
    tc_first = _make_tc_layer(False)
    tc_mid = _make_tc_layer(True)

    def row(v):
        return v.reshape(1, D)

    agg_r, agg_l = sc_agg(x_s, x_t, g_r, s_r, g_l, s_l, zeros)
    left, right, hl, hr = tc_first(
        agg_r, agg_l, x_t, x_s,
        W_rel0_lr, W_root0_lr, W_rel0_rl, W_root0_rl,
        row(b_rel0_lr), row(b_rel0_rl),
        row(ln_g_l[0]), row(ln_b_l[0]), row(ln_g_r[0]), row(ln_b_r[0]))

    for i in range(mid):
        agg_r, agg_l = sc_agg(hl, hr, g_r, s_r, g_l, s_l, zeros)
        j = min(i + 1, mid - 1)
        left, right, hl, hr = tc_mid(
            agg_r, agg_l, hr, hl, left, right,
            Wrel_lr_mid[i], Wroot_lr_mid[i], Wrel_rl_mid[i], Wroot_rl_mid[i],
            row(brel_lr_mid[i]), row(brel_rl_mid[i]),
            row(ln_g_l[j]), row(ln_b_l[j]), row(ln_g_r[j]), row(ln_b_r[j]))

    agg_r, agg_l = sc_agg(left, right, g_r, s_r, g_l, s_l, zeros)
    wpad = lambda w: jnp.pad(w, ((0, 0), (0, D - w.shape[1])))
    bpad = lambda b: jnp.pad(b, (0, D - b.shape[0])).reshape(1, D)
    l_out, r_out = _tc_final(
        agg_r, agg_l, left, right, x_s, x_t,
        wpad(Wrel_last_lr), wpad(Wroot_last_lr),
        wpad(Wrel_last_rl), wpad(Wroot_last_rl),
        bpad(brel_last_lr), bpad(brel_last_rl))
    return l_out[:, :3], r_out[:, :3]


# speedup vs baseline: 1.3285x; 1.3285x over previous
import functools

import jax
import jax.numpy as jnp
from jax import lax
from jax.experimental import pallas as pl
from jax.experimental.pallas import tpu as pltpu
from jax.experimental.pallas import tpu_sc as plsc

N = 10000
D = 128
CHUNK = 96
NSLOT = 4
N_SUBCORES = 16
SLAB = 624
SLAB_CHUNKS = (96, 96, 96, 96, 96, 96, 48)
TAIL0 = SLAB * N_SUBCORES
N_TC = 10240


def _make_sc_agg(ch_per_tile):
    assert ch_per_tile % NSLOT == 0
    nq = ch_per_tile // NSLOT
    mesh = plsc.VectorSubcoreMesh(core_axis_name="c", subcore_axis_name="s")

    def body(a_hbm, b_hbm, gr_hbm, sr_hbm, gl_hbm, sl_hbm, zeros_hbm,
             outr_hbm, outl_hbm,
             gidx, sidx, rows, agg,
             is0, is1, is2, is3, gs0, gs1, gs2, gs3):
        c = lax.axis_index("c")
        s = lax.axis_index("s")
        r0 = pl.multiple_of(s * SLAB, 8)
        isems = (is0, is1, is2, is3)
        gsems = (gs0, gs1, gs2, gs3)

        def run_dir(h_hbm, g_hbm, s_hbm, out_hbm):
            pltpu.sync_copy(zeros_hbm, rows.at[0])
            off = 0
            for ck in SLAB_CHUNKS:
                pltpu.sync_copy(rows.at[0, pl.ds(0, ck)],
                                agg.at[pl.ds(r0 + off, ck)])
                off += ck

            @pl.when(s == N_SUBCORES - 1)
            def _():
                pltpu.sync_copy(rows.at[0, pl.ds(0, N - TAIL0)],
                                agg.at[pl.ds(TAIL0, N - TAIL0)])

            plsc.subcore_barrier()

            base = s * ch_per_tile

            def fire_idx(j, m):
                eoff = pl.multiple_of((base + j) * CHUNK, 8)
                pltpu.async_copy(g_hbm.at[pl.ds(eoff, CHUNK)], gidx.at[m],
                                 isems[m])
                pltpu.async_copy(s_hbm.at[pl.ds(eoff, CHUNK)], sidx.at[m],
                                 isems[m])

            def wait_idx(m):
                pltpu.make_async_copy(g_hbm.at[pl.ds(0, CHUNK)], gidx.at[m],
                                      isems[m]).wait()
                pltpu.make_async_copy(s_hbm.at[pl.ds(0, CHUNK)], sidx.at[m],
                                      isems[m]).wait()

            def fire_gather(m):
                pltpu.async_copy(h_hbm.at[gidx.at[m]], rows.at[m], gsems[m])

            def wait_gather(m):
                pltpu.make_async_copy(h_hbm.at[gidx.at[m]], rows.at[m],
                                      gsems[m]).wait()

            fire_idx(0, 0)
            fire_idx(1, 1)
            fire_idx(2, 2)
            wait_idx(0)
            fire_gather(0)
            wait_idx(1)
            fire_gather(1)

            def qbody(jj, carry):
                jbase = jj * NSLOT
                for q in range(NSLOT):
                    j = jbase + q
                    m0 = q
                    m2 = (q + 2) % NSLOT
                    m3 = (q + 3) % NSLOT

                    @pl.when(j + 3 < ch_per_tile)
                    def _():
                        fire_idx(j + 3, m3)

                    @pl.when(j + 2 < ch_per_tile)
                    def _():
                        wait_idx(m2)
                        fire_gather(m2)

                    wait_gather(m0)
                    pltpu.sync_copy(rows.at[m0], agg.at[sidx.at[m0]],
                                    add=True)
                return carry

            lax.fori_loop(0, nq, qbody, 0, unroll=False)
            plsc.subcore_barrier()

            off = 0
            for ck in SLAB_CHUNKS:
                pltpu.sync_copy(agg.at[pl.ds(r0 + off, ck)],
                                rows.at[0, pl.ds(0, ck)])
                pltpu.sync_copy(rows.at[0, pl.ds(0, ck)],
                                out_hbm.at[pl.ds(r0 + off, ck)])
                off += ck

            @pl.when(s == N_SUBCORES - 1)
            def _():
                pltpu.sync_copy(agg.at[pl.ds(TAIL0, N - TAIL0)],
                                rows.at[1, pl.ds(0, N - TAIL0)])
                pltpu.sync_copy(rows.at[1, pl.ds(0, N - TAIL0)],
                                out_hbm.at[pl.ds(TAIL0, N - TAIL0)])

        @pl.when(c == 0)
        def _():
            run_dir(a_hbm, gr_hbm, sr_hbm, outr_hbm)

        @pl.when(c == 1)
        def _():
            run_dir(b_hbm, gl_hbm, sl_hbm, outl_hbm)

    return pl.kernel(
        body,
        out_type=(jax.ShapeDtypeStruct((N, D), jnp.float32),
                  jax.ShapeDtypeStruct((N, D), jnp.float32)),
        mesh=mesh,
        scratch_types=[
            pltpu.VMEM((NSLOT, CHUNK), jnp.int32),
            pltpu.VMEM((NSLOT, CHUNK), jnp.int32),
            pltpu.VMEM((NSLOT, CHUNK, D), jnp.float32),
            pltpu.VMEM_SHARED((N, D), jnp.float32),
        ] + [pltpu.SemaphoreType.DMA] * 8,
    )


BLK = 2048
GRID = N_TC // BLK


def _ln_relu(x, g, b):
    m = jnp.mean(x, axis=-1, keepdims=True)
    v = jnp.mean((x - m) ** 2, axis=-1, keepdims=True)
    return jax.nn.relu((x - m) / jnp.sqrt(v + 1e-5) * g + b)


def _dense_body(residual, aggr, aggl, rootr, rootl, left, right,
                wrel_lr, wroot_lr, wrel_rl, wroot_rl, brel_lr, brel_rl,
                gl, bl, gr, br,
                left_o, right_o, hl_o, hr_o):
    i = pl.program_id(0)
    row = lax.broadcasted_iota(jnp.int32, (BLK, 1), 0) + i * BLK
    valid = row < N
    r_new = (jnp.dot(aggr[...], wrel_lr[...], preferred_element_type=jnp.float32)
             + brel_lr[...]
             + jnp.dot(rootr[...], wroot_lr[...], preferred_element_type=jnp.float32))
    l_new = (jnp.dot(aggl[...], wrel_rl[...], preferred_element_type=jnp.float32)
             + brel_rl[...]
             + jnp.dot(rootl[...], wroot_rl[...], preferred_element_type=jnp.float32))
    if residual:
        l_new = left[...] + l_new
        r_new = right[...] + r_new
    zero = jnp.zeros_like(l_new)
    left_o[...] = jnp.where(valid, l_new, zero)
    right_o[...] = jnp.where(valid, r_new, zero)
    hl_o[...] = jnp.where(valid, _ln_relu(l_new, gl[...], bl[...]), zero)
    hr_o[...] = jnp.where(valid, _ln_relu(r_new, gr[...], br[...]), zero)


def _row_spec(blk=BLK):
    return pl.BlockSpec((blk, D), lambda i: (i, 0))


def _w_spec():
    return pl.BlockSpec((D, D), lambda i: (0, 0))


def _v_spec():
    return pl.BlockSpec((1, D), lambda i: (0, 0))


def _make_tc_layer(residual):
    n_row_in = 6 if residual else 4
    in_specs = ([_row_spec()] * n_row_in
                + [_w_spec()] * 4
                + [_v_spec()] * 6)
    if residual:
        body = functools.partial(_dense_body, True)
    else:
        def body(aggr, aggl, rootr, rootl, *rest):
            return _dense_body(False, aggr, aggl, rootr, rootl, None, None,
                               *rest)
    return pl.pallas_call(
        body,
        grid=(GRID,),
        in_specs=in_specs,
        out_specs=[_row_spec()] * 4,
        out_shape=[jax.ShapeDtypeStruct((N_TC, D), jnp.float32)] * 4,
    )


FBLK = 2000


def _final_body(aggr, aggl, left, right, xs, xt,
                wrel_lr, wroot_lr, wrel_rl, wroot_rl, brel_lr, brel_rl,
                l_o, r_o):
    r_new = (jnp.dot(aggr[...], wrel_lr[...], preferred_element_type=jnp.float32)
             + brel_lr[...]
             + jnp.dot(right[...], wroot_lr[...], preferred_element_type=jnp.float32))
    l_new = (jnp.dot(aggl[...], wrel_rl[...], preferred_element_type=jnp.float32)
             + brel_rl[...]
             + jnp.dot(left[...], wroot_rl[...], preferred_element_type=jnp.float32))

    def knowledge(v, flags):
        nrm = jnp.sqrt(jnp.sum(v * v, axis=-1, keepdims=True))
        v = v / jnp.maximum(nrm, 1e-12) * 10.0
        col = lax.broadcasted_iota(jnp.int32, v.shape, 1)
        lo = (jnp.abs(flags[:, 125:126]) > 0) & (col == 0)
        hi = (jnp.abs(flags[:, 127:128]) > 0) & (col == 2)
        return v - jnp.where(lo | hi, 10.0, 0.0)

    l_o[...] = knowledge(l_new, xs[...])
    r_o[...] = knowledge(r_new, xt[...])


_tc_final = pl.pallas_call(
    _final_body,
    grid=(N // FBLK,),
    in_specs=[_row_spec(FBLK)] * 6 + [_w_spec()] * 4 + [_v_spec()] * 2,
    out_specs=[_row_spec(FBLK)] * 2,
    out_shape=[jax.ShapeDtypeStruct((N, D), jnp.float32)] * 2,
)


def kernel(x_s, x_t, edge_index,
           W_rel0_lr, b_rel0_lr, W_root0_lr, W_rel0_rl, b_rel0_rl, W_root0_rl,
           Wrel_lr_mid, brel_lr_mid, Wroot_lr_mid, Wrel_rl_mid, brel_rl_mid,
           Wroot_rl_mid, ln_g_l, ln_b_l, ln_g_r, ln_b_r,
           Wrel_last_lr, brel_last_lr, Wroot_last_lr, Wrel_last_rl,
           brel_last_rl, Wroot_last_rl):
    E = edge_index.shape[1]
    mid = Wrel_lr_mid.shape[0]

    per_tile_e = -(-E // N_SUBCORES)
    ch_per_tile = -(-per_tile_e // CHUNK)
    ch_per_tile = -(-ch_per_tile // NSLOT) * NSLOT
    pad_e = ch_per_tile * N_SUBCORES * CHUNK

    src = edge_index[0]
    dst = edge_index[1]
    zpad_g = jnp.full((pad_e - E,), N, jnp.int32)
    zpad_s = jnp.zeros((pad_e - E,), jnp.int32)
    g_r = jnp.concatenate([src, zpad_g])
    s_r = jnp.concatenate([dst, zpad_s])
    g_l = jnp.concatenate([dst, zpad_g])
    s_l = jnp.concatenate([src, zpad_s])
    zeros = jnp.zeros((CHUNK, D), jnp.float32)
    rowpad = jnp.zeros((16, D), jnp.float32)
    xs_p = jnp.concatenate([x_s, rowpad])
    xt_p = jnp.concatenate([x_t, rowpad])

    sc_agg = _make_sc_agg(ch_per_tile)
    tc_first = _make_tc_layer(False)
    tc_mid = _make_tc_layer(True)

    def row(v):
        return v.reshape(1, D)

    agg_r, agg_l = sc_agg(xs_p, xt_p, g_r, s_r, g_l, s_l, zeros)
    left, right, hl, hr = tc_first(
        agg_r, agg_l, x_t, x_s,
        W_rel0_lr, W_root0_lr, W_rel0_rl, W_root0_rl,
        row(b_rel0_lr), row(b_rel0_rl),
        row(ln_g_l[0]), row(ln_b_l[0]), row(ln_g_r[0]), row(ln_b_r[0]))

    for i in range(mid):
        agg_r, agg_l = sc_agg(hl, hr, g_r, s_r, g_l, s_l, zeros)
        j = min(i + 1, mid - 1)
        left, right, hl, hr = tc_mid(
            agg_r, agg_l, hr, hl, left, right,
            Wrel_lr_mid[i], Wroot_lr_mid[i], Wrel_rl_mid[i], Wroot_rl_mid[i],
            row(brel_lr_mid[i]), row(brel_rl_mid[i]),
            row(ln_g_l[j]), row(ln_b_l[j]), row(ln_g_r[j]), row(ln_b_r[j]))

    agg_r, agg_l = sc_agg(left, right, g_r, s_r, g_l, s_l, zeros)
    wpad = lambda w: jnp.pad(w, ((0, 0), (0, D - w.shape[1])))
    bpad = lambda b: jnp.pad(b, (0, D - b.shape[0])).reshape(1, D)
    l_out, r_out = _tc_final(
        agg_r, agg_l, left, right, x_s, x_t,
        wpad(Wrel_last_lr), wpad(Wroot_last_lr),
        wpad(Wrel_last_rl), wpad(Wroot_last_rl),
        bpad(brel_last_lr), bpad(brel_last_rl))
    return l_out[:, :3], r_out[:, :3]
